# baseline (speedup 1.0000x reference)
import functools

import jax
import jax.numpy as jnp
from jax import lax
from jax.experimental import pallas as pl
from jax.experimental.pallas import tpu as pltpu
from jax.experimental.pallas import tpu_sc as plsc


@functools.cache
def _make_gather(B, D):
    info = plsc.get_sparse_core_info()
    NC, NS = info.num_cores, info.num_subcores
    NW = NC * NS
    assert B % NW == 0
    b_per_w = B // NW
    C = 1600
    assert b_per_w % C == 0
    n_chunks = b_per_w // C
    mesh = plsc.VectorSubcoreMesh(core_axis_name="c", subcore_axis_name="s")

    @functools.partial(
        pl.kernel,
        mesh=mesh,
        out_type=jax.ShapeDtypeStruct((B, D), jnp.float32),
        scratch_types=[
            pltpu.VMEM((n_chunks, C), jnp.int32),
            pltpu.VMEM((2, C, D), jnp.float32),
            pltpu.SemaphoreType.DMA((2,)),
            pltpu.SemaphoreType.DMA((2,)),
        ],
        compiler_params=pltpu.CompilerParams(use_tc_tiling_on_sc=False),
    )
    def k(idx_hbm, table_hbm, out_hbm, idx_v, rows_v, gsem, ssem):
        wid = lax.axis_index("s") * NC + lax.axis_index("c")
        base = wid * b_per_w
        pltpu.sync_copy(idx_hbm.at[wid], idx_v)

        gathers = [None, None]
        stores = [None, None]
        K = 4
        S = C // K

        def start_gather(i):
            b = i % 2
            cps = []
            for j in range(K):
                cp = pltpu.make_async_copy(
                    table_hbm.at[idx_v.at[i].at[pl.ds(j * S, S)]],
                    rows_v.at[b].at[pl.ds(j * S, S)],
                    gsem.at[b])
                cp.start()
                cps.append(cp)
            gathers[b] = cps

        start_gather(0)
        for i in range(n_chunks):
            b = i % 2
            if i + 1 < n_chunks:
                nb = (i + 1) % 2
                if stores[nb] is not None:
                    stores[nb].wait()
                    stores[nb] = None
                start_gather(i + 1)
            for cp in gathers[b]:
                cp.wait()
            cp = pltpu.make_async_copy(
                rows_v.at[b], out_hbm.at[pl.ds(base + i * C, C)], ssem.at[b])
            cp.start()
            stores[b] = cp
        for s in stores:
            if s is not None:
                s.wait()

    return k


def kernel(input_tensor, table):
    bt, h = input_tensor.shape
    v, d = table.shape
    b = bt * h
    info = plsc.get_sparse_core_info()
    nw = info.num_cores * info.num_subcores
    c = 1600
    idx = input_tensor.reshape(nw, (b // nw) // c, c).astype(jnp.int32)
    out = _make_gather(b, d)(idx, table)
    return out.reshape(bt, h, d)


# speedup vs baseline: 1.4834x; 1.4834x over previous
import functools

import jax
import jax.numpy as jnp
from jax import lax
from jax.experimental import pallas as pl
from jax.experimental.pallas import tpu as pltpu
from jax.experimental.pallas import tpu_sc as plsc

_L = 16


@functools.cache
def _make_gather(BT, H, D):
    info = plsc.get_sparse_core_info()
    NC, NS = info.num_cores, info.num_subcores
    NW = NC * NS
    assert BT % (NW * 128) == 0 and D % 8 == 0 and H % 2 == 0
    W = BT // NW
    E1, B1 = D // 8, BT // 128
    WB = W // 128
    mesh = plsc.VectorSubcoreMesh(core_axis_name="c", subcore_axis_name="s")

    @functools.partial(
        pl.kernel,
        mesh=mesh,
        out_type=jax.ShapeDtypeStruct((H, E1, B1, 8, 128), jnp.float32),
        scratch_types=[
            pltpu.VMEM((H, W), jnp.int32),
            pltpu.VMEM((2, W, D), jnp.float32),
            pltpu.VMEM((2, E1, WB, 8, 128), jnp.float32),
            pltpu.SemaphoreType.DMA((2,)),
            pltpu.SemaphoreType.DMA((2,)),
        ],
        compiler_params=pltpu.CompilerParams(
            use_tc_tiling_on_sc=False, needs_layout_passes=False),
    )
    def k(idx_hbm, table_hbm, y_hbm, idxT_v, rows_v, rowsT_v, gsem, ssem):
        wid = lax.axis_index("s") * NC + lax.axis_index("c")
        pltpu.sync_copy(idx_hbm.at[:, pl.ds(wid * W, W)], idxT_v)
        lanes = lax.iota(jnp.int32, _L)

        def gather_cp(h, b):
            return pltpu.make_async_copy(
                table_hbm.at[idxT_v.at[h]], rows_v.at[b], gsem.at[b])

        def store_cp(h, b):
            return pltpu.make_async_copy(
                rowsT_v.at[b], y_hbm.at[h, :, pl.ds(wid * WB, WB)], ssem.at[b])

        def transpose_rows(b):
            def body(t, carry):
                e = t // (W // _L)
                r, e0 = e // 8, e % 8
                b16 = (t % (W // _L)) * _L
                c, o = b16 // 128, b16 % 128
                vec = plsc.load_gather(
                    rows_v.at[b], [b16 + lanes, jnp.full((_L,), e, jnp.int32)])
                rowsT_v[b, r, c, e0, pl.ds(o, _L)] = vec
                return carry
            lax.fori_loop(0, D * (W // _L), body, 0, unroll=8)

        gather_cp(0, 0).start()

        def half(t, b):
            h = 2 * t + b
            gather_cp(h, b).wait()
            if b == 0:
                gather_cp(h + 1, 1 - b).start()
            else:
                @pl.when(t < H // 2 - 1)
                def _():
                    gather_cp(h + 1, 1 - b).start()

            @pl.when(t > 0)
            def _():
                store_cp(h - 2, b).wait()
            transpose_rows(b)
            store_cp(h, b).start()

        def body(t, carry):
            half(t, 0)
            half(t, 1)
            return carry

        lax.fori_loop(0, H // 2, body, 0)
        store_cp(H - 2, 0).wait()
        store_cp(H - 1, 1).wait()

    return k


def kernel(input_tensor, table):
    bt, h = input_tensor.shape
    v, d = table.shape
    y = _make_gather(bt, h, d)(input_tensor.T, table)
    return y.transpose(2, 4, 0, 1, 3).reshape(bt, h, d)
